# baseline (speedup 1.0000x reference)
import functools

import jax
import jax.numpy as jnp
from jax import lax
from jax.experimental import pallas as pl
from jax.experimental.pallas import tpu as pltpu
from jax.experimental.pallas import tpu_sc as plsc

N = 50000
D = 256
B = 128
BLK = 5000
NBLK = N // BLK

NW = 16
NJ = 25
LW = 128
PW = NJ * LW
NP = NW * PW
SEG = 144


def _tc_body(idx_ref, pred_ref, tgt_ref, rs_ref, cnt_ref, acc_ref):
    step = pl.program_id(0)

    @pl.when(step == 0)
    def _init():
        acc_ref[...] = jnp.zeros_like(acc_ref)

    diff = pred_ref[...] - tgt_ref[...]
    rs_ref[...] = jnp.sum(diff * diff, axis=1).reshape(1, 1, BLK)

    idx = idx_ref[0, 0, :]
    col_ids = jax.lax.broadcasted_iota(jnp.int32, (BLK, B), 1)
    onehot = jnp.where(idx[:, None] == col_ids,
                       jnp.float32(1), jnp.float32(0)).astype(jnp.bfloat16)
    acc_ref[...] += jnp.dot(jnp.ones((1, BLK), jnp.bfloat16), onehot,
                            preferred_element_type=jnp.float32)

    @pl.when(step == NBLK - 1)
    def _fini():
        cnt_ref[...] = acc_ref[...]


def _tc_row_mse_and_counts(idx3, pred, target):
    return pl.pallas_call(
        _tc_body,
        grid=(NBLK,),
        in_specs=[
            pl.BlockSpec((1, 1, BLK), lambda i: (i, 0, 0)),
            pl.BlockSpec((BLK, D), lambda i: (i, 0)),
            pl.BlockSpec((BLK, D), lambda i: (i, 0)),
        ],
        out_specs=[
            pl.BlockSpec((1, 1, BLK), lambda i: (i, 0, 0)),
            pl.BlockSpec((1, B), lambda i: (0, 0)),
        ],
        out_shape=[
            jax.ShapeDtypeStruct((NBLK, 1, BLK), jnp.float32),
            jax.ShapeDtypeStruct((1, B), jnp.float32),
        ],
        scratch_shapes=[pltpu.VMEM((1, B), jnp.float32)],
    )(idx3, pred, target)


@functools.partial(
    pl.kernel,
    out_type=jax.ShapeDtypeStruct((16,), jnp.float32),
    mesh=plsc.VectorSubcoreMesh(
        core_axis_name="c", subcore_axis_name="s", num_cores=1),
    scratch_types=[
        pltpu.VMEM((NJ, LW), jnp.float32),
        pltpu.VMEM((NJ, LW), jnp.int32),
        pltpu.VMEM((SEG,), jnp.float32),
        pltpu.VMEM((NW * SEG,), jnp.float32),
        pltpu.VMEM((B,), jnp.float32),
        pltpu.VMEM((16,), jnp.float32),
        pltpu.VMEM((16,), jnp.float32),
        pltpu.VMEM((16,), jnp.int32),
        pltpu.VMEM_SHARED((NW * SEG,), jnp.float32),
        pltpu.VMEM_SHARED((16,), jnp.float32),
    ],
)
def _sc_segment_mean(rs_hbm, idx_hbm, cnt_hbm, out_hbm,
                     vals_v, idx_v, zero_v, red_v, cnt_v, out_v, tmp_v,
                     zidx_v, sh_sums, sh_res):
    w = lax.axis_index("s")
    pltpu.sync_copy(rs_hbm.at[w], vals_v)
    pltpu.sync_copy(idx_hbm.at[w], idx_v)

    off = w * SEG
    for j in range(SEG // 16):
        zero_v[pl.ds(j * 16, 16)] = jnp.zeros((16,), jnp.float32)
    pltpu.sync_copy(zero_v, sh_sums.at[pl.ds(off, SEG)])

    for j in range(NJ):
        for k in range(LW // 16):
            sl = pl.ds(k * 16, 16)
            idx_v[j, sl] = idx_v[j, sl] + off

    for j in range(NJ):
        pltpu.sync_copy(vals_v.at[j], sh_sums.at[idx_v.at[j]], add=True)

    plsc.subcore_barrier()

    @pl.when(w == 0)
    def _finish():
        pltpu.sync_copy(sh_sums, red_v)
        pltpu.sync_copy(cnt_hbm.at[0], cnt_v)
        tot = jnp.zeros((16,), jnp.float32)
        for j in range(B // 16):
            s_j = jnp.zeros((16,), jnp.float32)
            for ww in range(NW):
                s_j = s_j + red_v[pl.ds(ww * SEG + j * 16, 16)]
            c_j = cnt_v[pl.ds(j * 16, 16)]
            tot = tot + s_j / jnp.maximum(c_j, 1.0)
        tmp_v[...] = tot / jnp.float32(D * B)
        zidx_v[...] = jnp.zeros((16,), jnp.int32)
        out_v[...] = jnp.zeros((16,), jnp.float32)
        pltpu.sync_copy(out_v, sh_res)
        pltpu.sync_copy(tmp_v, sh_res.at[zidx_v], add=True)
        pltpu.sync_copy(sh_res, out_v)
        pltpu.sync_copy(out_v, out_hbm)


def kernel(pred, target, batch_idx, batch_size):
    del batch_size
    idx32 = batch_idx.astype(jnp.int32)
    idx3 = idx32.reshape(NBLK, 1, BLK)
    rs, cnt = _tc_row_mse_and_counts(idx3, pred, target)
    rs_pad = jnp.concatenate(
        [rs.reshape(N), jnp.zeros((NP - N,), jnp.float32)]
    ).reshape(NW, NJ, LW)
    idx_pad = jnp.concatenate(
        [idx32, jnp.full((NP - N,), B, jnp.int32)]).reshape(NW, NJ, LW)
    out = _sc_segment_mean(rs_pad, idx_pad, cnt)
    return out[0]


# speedup vs baseline: 1.2757x; 1.2757x over previous
import functools

import jax
import jax.numpy as jnp
from jax import lax
from jax.experimental import pallas as pl
from jax.experimental.pallas import tpu as pltpu
from jax.experimental.pallas import tpu_sc as plsc

N = 50000
D = 256
B = 128
BLK = 5000
NBLK = N // BLK

NW = 16
NJ = 25
LW = 128
PW = NJ * LW
NP = NW * PW
SEG = 144


def _tc_body(idx_ref, pred_ref, tgt_ref, out_ref, acc_ref):
    step = pl.program_id(0)

    @pl.when(step == 0)
    def _init():
        acc_ref[...] = jnp.zeros_like(acc_ref)

    diff = pred_ref[...] - tgt_ref[...]
    sqb = (diff * diff).astype(jnp.bfloat16)
    idx = idx_ref[0, 0, :]
    row_ids = jax.lax.broadcasted_iota(jnp.int32, (B, BLK), 0)
    onehot_t = jnp.where(row_ids == idx[None, :],
                         jnp.float32(1), jnp.float32(0)
                         ).astype(jnp.bfloat16)
    acc_ref[...] += jnp.dot(onehot_t, sqb,
                            preferred_element_type=jnp.float32)

    @pl.when(step == NBLK - 1)
    def _fini():
        out_ref[...] = acc_ref[...]


def _tc_seg_sums(idx3, pred, target):
    return pl.pallas_call(
        _tc_body,
        grid=(NBLK,),
        in_specs=[
            pl.BlockSpec((1, 1, BLK), lambda i: (i, 0, 0)),
            pl.BlockSpec((BLK, D), lambda i: (i, 0)),
            pl.BlockSpec((BLK, D), lambda i: (i, 0)),
        ],
        out_specs=pl.BlockSpec((B, D), lambda i: (0, 0)),
        out_shape=jax.ShapeDtypeStruct((B, D), jnp.float32),
        scratch_shapes=[pltpu.VMEM((B, D), jnp.float32)],
    )(idx3, pred, target)


@functools.partial(
    pl.kernel,
    out_type=jax.ShapeDtypeStruct((B,), jnp.float32),
    mesh=plsc.VectorSubcoreMesh(
        core_axis_name="c", subcore_axis_name="s", num_cores=1),
    scratch_types=[
        pltpu.VMEM((NJ, LW), jnp.int32),
        pltpu.VMEM((LW,), jnp.float32),
        pltpu.VMEM((SEG,), jnp.float32),
        pltpu.VMEM((NW * SEG,), jnp.float32),
        pltpu.VMEM((B,), jnp.float32),
        pltpu.VMEM_SHARED((NW * SEG,), jnp.float32),
    ],
)
def _sc_counts(idx_hbm, out_hbm, idx_v, ones_v, zero_v, red_v, out_v,
               sh_cnts):
    w = lax.axis_index("s")
    pltpu.sync_copy(idx_hbm.at[w], idx_v)

    for k in range(LW // 16):
        ones_v[pl.ds(k * 16, 16)] = jnp.ones((16,), jnp.float32)
    for j in range(SEG // 16):
        zero_v[pl.ds(j * 16, 16)] = jnp.zeros((16,), jnp.float32)

    off = w * SEG
    pltpu.sync_copy(zero_v, sh_cnts.at[pl.ds(off, SEG)])

    for j in range(NJ):
        for k in range(LW // 16):
            sl = pl.ds(k * 16, 16)
            idx_v[j, sl] = idx_v[j, sl] + off

    for j in range(NJ):
        pltpu.sync_copy(ones_v, sh_cnts.at[idx_v.at[j]], add=True)

    plsc.subcore_barrier()

    @pl.when(w == 0)
    def _finish():
        pltpu.sync_copy(sh_cnts, red_v)
        for j in range(B // 16):
            c_j = jnp.zeros((16,), jnp.float32)
            for ww in range(NW):
                c_j = c_j + red_v[pl.ds(ww * SEG + j * 16, 16)]
            out_v[pl.ds(j * 16, 16)] = c_j
        pltpu.sync_copy(out_v, out_hbm)


def _combine_body(acc_ref, cnt_ref, out_ref):
    seg = jnp.sum(acc_ref[...], axis=1)
    cnt = cnt_ref[...]
    tot = jnp.sum(seg / jnp.maximum(cnt, 1.0))
    out_ref[...] = (tot / (D * B)).reshape(1, 1)


def _tc_combine(acc, cnt):
    return pl.pallas_call(
        _combine_body,
        out_shape=jax.ShapeDtypeStruct((1, 1), jnp.float32),
    )(acc, cnt)


def kernel(pred, target, batch_idx, batch_size):
    del batch_size
    idx32 = batch_idx.astype(jnp.int32)
    idx3 = idx32.reshape(NBLK, 1, BLK)
    idx_pad = jnp.concatenate(
        [idx32, jnp.full((NP - N,), B, jnp.int32)]).reshape(NW, NJ, LW)
    acc = _tc_seg_sums(idx3, pred, target)
    cnt = _sc_counts(idx_pad)
    out = _tc_combine(acc, cnt)
    return out[0, 0]
